# baseline (speedup 1.0000x reference)
import jax
import jax.numpy as jnp
from jax.experimental import pallas as pl
from jax.experimental.pallas import tpu as pltpu

_MIB = 1 << 20


def _gcn_kernel(x_ref, adj_ref, w1_ref, b1_ref, w2_ref, b2_ref, o_ref,
                s2_ref):
    N = adj_ref.shape[0]
    half = N // 2
    phase = pl.program_id(1)

    @pl.when(phase == 0)
    def _layer1():
        x = x_ref[...]
        w1 = w1_ref[...]
        w2 = w2_ref[...]

        def chain(rows):
            ax = jnp.dot(adj_ref[rows, :], x,
                         preferred_element_type=jnp.float32)
            h1 = jnp.dot(ax, w1, preferred_element_type=jnp.float32)
            h1 = jnp.maximum(h1 + b1_ref[...], 0.0)
            s2_ref[rows, :] = jnp.dot(h1, w2,
                                      preferred_element_type=jnp.float32)

        chain(pl.ds(0, half))
        chain(pl.ds(half, half))

    @pl.when(phase == 1)
    def _layer2():
        s2 = s2_ref[...]

        def chain(rows):
            out = jnp.dot(adj_ref[rows, :], s2,
                          preferred_element_type=jnp.float32)
            o_ref[rows, :] = jnp.tanh(out + b2_ref[...]).astype(o_ref.dtype)

        chain(pl.ds(0, half))
        chain(pl.ds(half, half))


def kernel(x, adj, w1, b1, w2, b2):
    B, N, nfeat = x.shape
    nhid = w1.shape[1]
    nclass = w2.shape[1]

    b1_2d = b1.reshape(1, nhid)
    b2_2d = b2.reshape(1, nclass)

    wspec = lambda shape: pl.BlockSpec(shape, lambda b, p: (0,) * len(shape))
    return pl.pallas_call(
        _gcn_kernel,
        out_shape=jax.ShapeDtypeStruct((B, N, nclass), x.dtype),
        grid_spec=pltpu.PrefetchScalarGridSpec(
            num_scalar_prefetch=0,
            grid=(B, 2),
            in_specs=[
                pl.BlockSpec((None, N, nfeat), lambda b, p: (b, 0, 0)),
                pl.BlockSpec((None, N, N), lambda b, p: (b, 0, 0)),
                wspec((nfeat, nhid)),
                wspec((1, nhid)),
                wspec((nhid, nclass)),
                wspec((1, nclass)),
            ],
            out_specs=pl.BlockSpec((None, N, nclass), lambda b, p: (b, 0, 0)),
            scratch_shapes=[pltpu.VMEM((N, nclass), jnp.float32)],
        ),
        compiler_params=pltpu.CompilerParams(
            dimension_semantics=("arbitrary", "arbitrary"),
            vmem_limit_bytes=64 * _MIB,
        ),
    )(x, adj, w1, b1_2d, w2, b2_2d)
